# baseline (speedup 1.0000x reference)
import functools

import jax
import jax.numpy as jnp
from jax.experimental import pallas as pl
from jax.experimental.pallas import tpu as pltpu

_E = 8
_K = 2
_D = 1024
_B = 4096
_SWITCHLOSS = 0.01
_ZLOSS = 0.001
_LANES = 128

_BM_G = 512
_NB_G = _B // _BM_G
_BM_X = 512
_NB_X = _B // _BM_X


def _gating_body(x_ref, wgt_ref, gates_ref, loss_ref, psum_ref, freq_ref,
                 zsum_ref):
    i = pl.program_id(0)
    lanes = jax.lax.broadcasted_iota(jnp.int32, (_BM_G, _LANES), 1)
    emask = lanes < _E

    logits = jnp.dot(x_ref[...], wgt_ref[...],
                     preferred_element_type=jnp.float32)
    logits = jnp.where(emask, logits, -1e30)
    m = jnp.max(logits, axis=1, keepdims=True)
    p = jnp.exp(logits - m)
    s = jnp.sum(p, axis=1, keepdims=True)
    probs = p / s
    lse = jnp.log(s) + m

    g1 = jnp.max(probs, axis=1, keepdims=True)
    i1 = jnp.min(jnp.where(probs == g1, lanes, _LANES), axis=1, keepdims=True)
    probs2 = jnp.where(lanes == i1, -1.0, probs)
    g2 = jnp.max(probs2, axis=1, keepdims=True)
    i2 = jnp.min(jnp.where(probs2 == g2, lanes, _LANES), axis=1, keepdims=True)
    denom = g1 + g2 + 1e-6
    g1n = g1 / denom
    g2n = g2 / denom
    gates = (jnp.where(lanes == i1, g1n, 0.0)
             + jnp.where(lanes == i2, g2n, 0.0))
    gates_ref[...] = gates

    psum_blk = jnp.sum(probs, axis=0, keepdims=True)
    freq_blk = jnp.sum((gates > 0.0).astype(jnp.float32), axis=0,
                       keepdims=True)
    z_blk = jnp.sum(lse * lse)

    @pl.when(i == 0)
    def _init():
        psum_ref[...] = psum_blk
        freq_ref[...] = freq_blk
        zsum_ref[0, 0] = z_blk

    @pl.when(i > 0)
    def _acc():
        psum_ref[...] += psum_blk
        freq_ref[...] += freq_blk
        zsum_ref[0, 0] += z_blk

    @pl.when(i == _NB_G - 1)
    def _finish():
        psum = psum_ref[...]
        pnorm = psum / jnp.sum(jnp.abs(psum))
        freqs = freq_ref[...]
        fnorm = freqs / jnp.sum(jnp.abs(freqs))
        switch = jnp.sum(pnorm * fnorm) * _E
        z = zsum_ref[0, 0] / _B
        loss = _SWITCHLOSS * switch + _ZLOSS * z
        loss_ref[...] = jnp.broadcast_to(loss, (1, _LANES))


def _gating(x, wgt_pad, interpret=False):
    return pl.pallas_call(
        _gating_body,
        grid=(_NB_G,),
        in_specs=[
            pl.BlockSpec((_BM_G, _D), lambda i: (i, 0)),
            pl.BlockSpec((_D, _LANES), lambda i: (0, 0)),
        ],
        out_specs=[
            pl.BlockSpec((_BM_G, _LANES), lambda i: (i, 0)),
            pl.BlockSpec((1, _LANES), lambda i: (0, 0)),
        ],
        out_shape=[
            jax.ShapeDtypeStruct((_B, _LANES), jnp.float32),
            jax.ShapeDtypeStruct((1, _LANES), jnp.float32),
        ],
        scratch_shapes=[
            pltpu.VMEM((1, _LANES), jnp.float32),
            pltpu.VMEM((1, _LANES), jnp.float32),
            pltpu.SMEM((1, 1), jnp.float32),
        ],
        interpret=interpret,
    )(x, wgt_pad)


def _expert_body(x_ref, we_ref, be_ref, gates_ref, out_ref):
    e = pl.program_id(1)
    lanes = jax.lax.broadcasted_iota(jnp.int32, (_BM_X, _LANES), 1)
    g_col = jnp.sum(jnp.where(lanes == e, gates_ref[...], 0.0), axis=1,
                    keepdims=True)
    y = jnp.dot(x_ref[...], we_ref[0],
                preferred_element_type=jnp.float32,
                precision=jax.lax.Precision.HIGHEST)
    contrib = g_col * (y + be_ref[0])

    @pl.when(e == 0)
    def _init():
        out_ref[...] = contrib

    @pl.when(e > 0)
    def _acc():
        out_ref[...] += contrib


def _experts(x, we, be3, gates, interpret=False):
    return pl.pallas_call(
        _expert_body,
        grid=(_NB_X, _E),
        in_specs=[
            pl.BlockSpec((_BM_X, _D), lambda i, e: (i, 0)),
            pl.BlockSpec((1, _D, _D), lambda i, e: (e, 0, 0)),
            pl.BlockSpec((1, 1, _D), lambda i, e: (e, 0, 0)),
            pl.BlockSpec((_BM_X, _LANES), lambda i, e: (i, 0)),
        ],
        out_specs=pl.BlockSpec((_BM_X, _D), lambda i, e: (i, 0)),
        out_shape=jax.ShapeDtypeStruct((_B, _D), jnp.float32),
        interpret=interpret,
    )(x, we, be3, gates)


@functools.partial(jax.jit, static_argnames=("interpret",))
def kernel(x, Wg, We, be, interpret=False):
    wgt_pad = jnp.zeros((_D, _LANES), jnp.float32).at[:, :_E].set(Wg.T)
    gates, loss_row = _gating(x, wgt_pad, interpret=interpret)
    be3 = be.reshape(_E, 1, _D)
    out = _experts(x, We, be3, gates, interpret=interpret)
    return out, loss_row[0, 0]


# speedup vs baseline: 3.1391x; 3.1391x over previous
import functools

import jax
import jax.numpy as jnp
from jax.experimental import pallas as pl
from jax.experimental.pallas import tpu as pltpu

_E = 8
_K = 2
_D = 1024
_B = 4096
_SWITCHLOSS = 0.01
_ZLOSS = 0.001
_LANES = 128

_BM_G = 512
_NB_G = _B // _BM_G
_BM_X = 512
_NB_X = _B // _BM_X


def _gating_body(x_ref, wgt_ref, gates_ref, loss_ref, psum_ref, freq_ref,
                 zsum_ref):
    i = pl.program_id(0)
    lanes = jax.lax.broadcasted_iota(jnp.int32, (_BM_G, _LANES), 1)
    emask = lanes < _E

    logits = jnp.dot(x_ref[...], wgt_ref[...],
                     preferred_element_type=jnp.float32)
    logits = jnp.where(emask, logits, -1e30)
    m = jnp.max(logits, axis=1, keepdims=True)
    p = jnp.exp(logits - m)
    s = jnp.sum(p, axis=1, keepdims=True)
    probs = p / s
    lse = jnp.log(s) + m

    g1 = jnp.max(probs, axis=1, keepdims=True)
    i1 = jnp.min(jnp.where(probs == g1, lanes, _LANES), axis=1, keepdims=True)
    probs2 = jnp.where(lanes == i1, -1.0, probs)
    g2 = jnp.max(probs2, axis=1, keepdims=True)
    i2 = jnp.min(jnp.where(probs2 == g2, lanes, _LANES), axis=1, keepdims=True)
    denom = g1 + g2 + 1e-6
    g1n = g1 / denom
    g2n = g2 / denom
    gates = (jnp.where(lanes == i1, g1n, 0.0)
             + jnp.where(lanes == i2, g2n, 0.0))
    gates_ref[...] = gates

    psum_blk = jnp.sum(probs, axis=0, keepdims=True)
    freq_blk = jnp.sum((gates > 0.0).astype(jnp.float32), axis=0,
                       keepdims=True)
    z_blk = jnp.sum(lse * lse)

    @pl.when(i == 0)
    def _init():
        psum_ref[...] = psum_blk
        freq_ref[...] = freq_blk
        zsum_ref[0, 0] = z_blk

    @pl.when(i > 0)
    def _acc():
        psum_ref[...] += psum_blk
        freq_ref[...] += freq_blk
        zsum_ref[0, 0] += z_blk

    @pl.when(i == _NB_G - 1)
    def _finish():
        psum = psum_ref[...]
        pnorm = psum / jnp.sum(jnp.abs(psum))
        freqs = freq_ref[...]
        fnorm = freqs / jnp.sum(jnp.abs(freqs))
        switch = jnp.sum(pnorm * fnorm) * _E
        z = zsum_ref[0, 0] / _B
        loss = _SWITCHLOSS * switch + _ZLOSS * z
        loss_ref[...] = jnp.broadcast_to(loss, (1, _LANES))


def _gating(x, wgt_pad, interpret=False):
    return pl.pallas_call(
        _gating_body,
        grid=(_NB_G,),
        in_specs=[
            pl.BlockSpec((_BM_G, _D), lambda i: (i, 0)),
            pl.BlockSpec((_D, _LANES), lambda i: (0, 0)),
        ],
        out_specs=[
            pl.BlockSpec((_BM_G, _LANES), lambda i: (i, 0)),
            pl.BlockSpec((1, _LANES), lambda i: (0, 0)),
        ],
        out_shape=[
            jax.ShapeDtypeStruct((_B, _LANES), jnp.float32),
            jax.ShapeDtypeStruct((1, _LANES), jnp.float32),
        ],
        scratch_shapes=[
            pltpu.VMEM((1, _LANES), jnp.float32),
            pltpu.VMEM((1, _LANES), jnp.float32),
            pltpu.SMEM((1, 1), jnp.float32),
        ],
        interpret=interpret,
    )(x, wgt_pad)


def _expert_body(x_ref, we_ref, be_ref, gates_ref, out_ref):
    e = pl.program_id(1)
    lanes = jax.lax.broadcasted_iota(jnp.int32, (_BM_X, _LANES), 1)
    g_col = jnp.sum(jnp.where(lanes == e, gates_ref[...], 0.0), axis=1,
                    keepdims=True)
    y = jnp.dot(x_ref[...], we_ref[0],
                preferred_element_type=jnp.float32)
    contrib = g_col * (y + be_ref[0])

    @pl.when(e == 0)
    def _init():
        out_ref[...] = contrib

    @pl.when(e > 0)
    def _acc():
        out_ref[...] += contrib


def _experts(x, we, be3, gates, interpret=False):
    return pl.pallas_call(
        _expert_body,
        grid=(_NB_X, _E),
        in_specs=[
            pl.BlockSpec((_BM_X, _D), lambda i, e: (i, 0)),
            pl.BlockSpec((1, _D, _D), lambda i, e: (e, 0, 0)),
            pl.BlockSpec((1, 1, _D), lambda i, e: (e, 0, 0)),
            pl.BlockSpec((_BM_X, _LANES), lambda i, e: (i, 0)),
        ],
        out_specs=pl.BlockSpec((_BM_X, _D), lambda i, e: (i, 0)),
        out_shape=jax.ShapeDtypeStruct((_B, _D), jnp.float32),
        interpret=interpret,
    )(x, we, be3, gates)


@functools.partial(jax.jit, static_argnames=("interpret",))
def kernel(x, Wg, We, be, interpret=False):
    wgt_pad = jnp.zeros((_D, _LANES), jnp.float32).at[:, :_E].set(Wg.T)
    gates, loss_row = _gating(x, wgt_pad, interpret=interpret)
    be3 = be.reshape(_E, 1, _D)
    out = _experts(x, We, be3, gates, interpret=interpret)
    return out, loss_row[0, 0]


# speedup vs baseline: 4.1233x; 1.3136x over previous
import functools

import jax
import jax.numpy as jnp
from jax.experimental import pallas as pl
from jax.experimental.pallas import tpu as pltpu

_E = 8
_K = 2
_D = 1024
_B = 4096
_SWITCHLOSS = 0.01
_ZLOSS = 0.001
_LANES = 128

_BM_G = 512
_NB_G = _B // _BM_G
_BM_X = 2048
_NB_X = _B // _BM_X


def _gating_body(x_ref, wgt_ref, gates_ref, loss_ref, psum_ref, freq_ref,
                 zsum_ref):
    i = pl.program_id(0)
    lanes = jax.lax.broadcasted_iota(jnp.int32, (_BM_G, _LANES), 1)
    emask = lanes < _E

    logits = jnp.dot(x_ref[...], wgt_ref[...],
                     preferred_element_type=jnp.float32)
    logits = jnp.where(emask, logits, -1e30)
    m = jnp.max(logits, axis=1, keepdims=True)
    p = jnp.exp(logits - m)
    s = jnp.sum(p, axis=1, keepdims=True)
    probs = p / s
    lse = jnp.log(s) + m

    g1 = jnp.max(probs, axis=1, keepdims=True)
    i1 = jnp.min(jnp.where(probs == g1, lanes, _LANES), axis=1, keepdims=True)
    probs2 = jnp.where(lanes == i1, -1.0, probs)
    g2 = jnp.max(probs2, axis=1, keepdims=True)
    i2 = jnp.min(jnp.where(probs2 == g2, lanes, _LANES), axis=1, keepdims=True)
    denom = g1 + g2 + 1e-6
    g1n = g1 / denom
    g2n = g2 / denom
    gates = (jnp.where(lanes == i1, g1n, 0.0)
             + jnp.where(lanes == i2, g2n, 0.0))
    gates_ref[...] = gates

    psum_blk = jnp.sum(probs, axis=0, keepdims=True)
    freq_blk = jnp.sum((gates > 0.0).astype(jnp.float32), axis=0,
                       keepdims=True)
    z_blk = jnp.sum(lse * lse)

    @pl.when(i == 0)
    def _init():
        psum_ref[...] = psum_blk
        freq_ref[...] = freq_blk
        zsum_ref[0, 0] = z_blk

    @pl.when(i > 0)
    def _acc():
        psum_ref[...] += psum_blk
        freq_ref[...] += freq_blk
        zsum_ref[0, 0] += z_blk

    @pl.when(i == _NB_G - 1)
    def _finish():
        psum = psum_ref[...]
        pnorm = psum / jnp.sum(jnp.abs(psum))
        freqs = freq_ref[...]
        fnorm = freqs / jnp.sum(jnp.abs(freqs))
        switch = jnp.sum(pnorm * fnorm) * _E
        z = zsum_ref[0, 0] / _B
        loss = _SWITCHLOSS * switch + _ZLOSS * z
        loss_ref[...] = jnp.broadcast_to(loss, (1, _LANES))


def _gating(x, wgt_pad, interpret=False):
    return pl.pallas_call(
        _gating_body,
        grid=(_NB_G,),
        in_specs=[
            pl.BlockSpec((_BM_G, _D), lambda i: (i, 0)),
            pl.BlockSpec((_D, _LANES), lambda i: (0, 0)),
        ],
        out_specs=[
            pl.BlockSpec((_BM_G, _LANES), lambda i: (i, 0)),
            pl.BlockSpec((1, _LANES), lambda i: (0, 0)),
        ],
        out_shape=[
            jax.ShapeDtypeStruct((_B, _LANES), jnp.float32),
            jax.ShapeDtypeStruct((1, _LANES), jnp.float32),
        ],
        scratch_shapes=[
            pltpu.VMEM((1, _LANES), jnp.float32),
            pltpu.VMEM((1, _LANES), jnp.float32),
            pltpu.SMEM((1, 1), jnp.float32),
        ],
        interpret=interpret,
    )(x, wgt_pad)


def _expert_body(x_ref, we_ref, be_ref, gates_ref, out_ref):
    e = pl.program_id(1)
    lanes = jax.lax.broadcasted_iota(jnp.int32, (_BM_X, _LANES), 1)
    g_col = jnp.sum(jnp.where(lanes == e, gates_ref[...], 0.0), axis=1,
                    keepdims=True)
    y = jnp.dot(x_ref[...], we_ref[0],
                preferred_element_type=jnp.float32)
    contrib = g_col * (y + be_ref[0])

    @pl.when(e == 0)
    def _init():
        out_ref[...] = contrib

    @pl.when(e > 0)
    def _acc():
        out_ref[...] += contrib


def _experts(x, we, be3, gates, interpret=False):
    return pl.pallas_call(
        _expert_body,
        grid=(_NB_X, _E),
        in_specs=[
            pl.BlockSpec((_BM_X, _D), lambda i, e: (i, 0)),
            pl.BlockSpec((1, _D, _D), lambda i, e: (e, 0, 0)),
            pl.BlockSpec((1, 1, _D), lambda i, e: (e, 0, 0)),
            pl.BlockSpec((_BM_X, _LANES), lambda i, e: (i, 0)),
        ],
        out_specs=pl.BlockSpec((_BM_X, _D), lambda i, e: (i, 0)),
        out_shape=jax.ShapeDtypeStruct((_B, _D), jnp.float32),
        interpret=interpret,
    )(x, we, be3, gates)


@functools.partial(jax.jit, static_argnames=("interpret",))
def kernel(x, Wg, We, be, interpret=False):
    wgt_pad = jnp.zeros((_D, _LANES), jnp.float32).at[:, :_E].set(Wg.T)
    gates, loss_row = _gating(x, wgt_pad, interpret=interpret)
    be3 = be.reshape(_E, 1, _D)
    out = _experts(x, We, be3, gates, interpret=interpret)
    return out, loss_row[0, 0]
